# baseline (device time: 264932 ns/iter reference)
import jax
import jax.numpy as jnp
from jax import lax
from jax.experimental import pallas as pl
from jax.experimental.pallas import tpu as pltpu

B, S, H_LOC, D = 4, 1024, 16, 128
K = H_LOC * D
N = 4096
S_HALF = S // 2
NC = 512
G = N // NC
ROWS = B * S_HALF


def kernel(O, Wo):
    O2 = O.reshape(B * S, K).astype(jnp.bfloat16)
    Wo2 = Wo.astype(jnp.bfloat16)

    def body(o_ref, wo_hbm, out_ref, wo_buf, send_buf, recv_buf, acc_buf,
             res_buf, wo_sem, out_sem, send_sems, recv_sems):
        my_x = lax.axis_index("x")
        my_y = lax.axis_index("y")
        my_z = lax.axis_index("z")
        peer = (my_x, 1 - my_y, my_z)

        barrier = pltpu.get_barrier_semaphore()
        pl.semaphore_signal(barrier, inc=1, device_id=peer,
                            device_id_type=pl.DeviceIdType.MESH)
        pl.semaphore_wait(barrier, 1)

        my_base = my_y * S_HALF
        peer_base = (1 - my_y) * S_HALF

        def wo_fetch(j):
            return pltpu.make_async_copy(
                wo_hbm.at[:, pl.ds(j * NC, NC)], wo_buf.at[j % 2],
                wo_sem.at[j % 2])

        wo_fetch(0).start()

        rdmas = [None] * G
        ocps = [None] * G
        for j in range(G):
            wslot = j % 2
            aslot = j % 2
            sslot = j % 2

            if j + 1 < G:
                wo_fetch(j + 1).start()
            wo_fetch(j).wait()

            if j >= 2:
                rdmas[j - 2].wait_send()
                ocps[j - 2].wait()
            for b in range(B):
                res_buf[...] = jnp.dot(
                    o_ref[b * S:(b + 1) * S, :], wo_buf[wslot],
                    preferred_element_type=jnp.float32)
                send_buf[sslot, b * S_HALF:(b + 1) * S_HALF, :] = (
                    res_buf[pl.ds(peer_base, S_HALF), :]
                    .astype(jnp.bfloat16))
                acc_buf[aslot, b * S_HALF:(b + 1) * S_HALF, :] = (
                    res_buf[pl.ds(my_base, S_HALF), :])

            rdmas[j] = pltpu.make_async_remote_copy(
                src_ref=send_buf.at[sslot],
                dst_ref=recv_buf.at[j],
                send_sem=send_sems.at[sslot],
                recv_sem=recv_sems.at[j],
                device_id=peer,
                device_id_type=pl.DeviceIdType.MESH,
            )
            rdmas[j].start()

            if j >= 1:
                k = j - 1
                rdmas[k].wait_recv()
                acc_buf[k % 2] = acc_buf[k % 2] + recv_buf[k].astype(
                    jnp.float32)
                ocps[k] = pltpu.make_async_copy(
                    acc_buf.at[k % 2], out_ref.at[:, pl.ds(k * NC, NC)],
                    out_sem.at[k % 2])
                ocps[k].start()

        k = G - 1
        rdmas[k].wait_recv()
        acc_buf[k % 2] = acc_buf[k % 2] + recv_buf[k].astype(jnp.float32)
        ocps[k] = pltpu.make_async_copy(
            acc_buf.at[k % 2], out_ref.at[:, pl.ds(k * NC, NC)],
            out_sem.at[k % 2])
        ocps[k].start()
        for j in (G - 2, G - 1):
            rdmas[j].wait_send()
            ocps[j].wait()

    out = pl.pallas_call(
        body,
        out_shape=jax.ShapeDtypeStruct((ROWS, N), jnp.float32),
        in_specs=[
            pl.BlockSpec(memory_space=pltpu.VMEM),
            pl.BlockSpec(memory_space=pl.ANY),
        ],
        out_specs=pl.BlockSpec(memory_space=pltpu.MemorySpace.HBM),
        scratch_shapes=[
            pltpu.VMEM((2, K, NC), jnp.bfloat16),
            pltpu.VMEM((2, ROWS, NC), jnp.bfloat16),
            pltpu.VMEM((G, ROWS, NC), jnp.bfloat16),
            pltpu.VMEM((2, ROWS, NC), jnp.float32),
            pltpu.VMEM((S, NC), jnp.float32),
            pltpu.SemaphoreType.DMA((2,)),
            pltpu.SemaphoreType.DMA((2,)),
            pltpu.SemaphoreType.DMA((2,)),
            pltpu.SemaphoreType.DMA((G,)),
        ],
        compiler_params=pltpu.CompilerParams(
            collective_id=0, vmem_limit_bytes=56 * 1024 * 1024),
    )(O2, Wo2)
    return out.reshape(B, S_HALF, N)


# device time: 264930 ns/iter; 1.0000x vs baseline; 1.0000x over previous
import jax
import jax.numpy as jnp
from jax import lax
from jax.experimental import pallas as pl
from jax.experimental.pallas import tpu as pltpu

B, S, H_LOC, D = 4, 1024, 16, 128
K = H_LOC * D
N = 4096
S_HALF = S // 2
NC = 512
G = N // NC
ROWS = B * S_HALF


def kernel(O, Wo):
    O2 = O.reshape(B * S, K).astype(jnp.bfloat16)
    Wo2 = Wo.astype(jnp.bfloat16)

    def body(o_ref, wo_hbm, out_ref, wo_buf, send_buf, recv_buf, acc_buf,
             res_buf, wo_sem, out_sem, send_sems, recv_sems):
        my_x = lax.axis_index("x")
        my_y = lax.axis_index("y")
        my_z = lax.axis_index("z")
        peer = (my_x, 1 - my_y, my_z)

        barrier = pltpu.get_barrier_semaphore()
        pl.semaphore_signal(barrier, inc=1, device_id=peer,
                            device_id_type=pl.DeviceIdType.MESH)
        pl.semaphore_wait(barrier, 1)

        my_base = my_y * S_HALF
        peer_base = (1 - my_y) * S_HALF

        def wo_fetch(j):
            return pltpu.make_async_copy(
                wo_hbm.at[:, pl.ds(j * NC, NC)], wo_buf.at[j % 2],
                wo_sem.at[j % 2])

        wo_fetch(0).start()

        rdmas = [None] * G
        ocps = [None] * G
        def finalize(k):
            rdmas[k].wait_recv()
            acc_buf[k % 3] = acc_buf[k % 3] + recv_buf[k].astype(jnp.float32)
            ocps[k] = pltpu.make_async_copy(
                acc_buf.at[k % 3], out_ref.at[:, pl.ds(k * NC, NC)],
                out_sem.at[k % 3])
            ocps[k].start()

        for j in range(G):
            wslot = j % 2
            aslot = j % 3
            sslot = j % 2

            if j + 1 < G:
                wo_fetch(j + 1).start()
            wo_fetch(j).wait()

            if j >= 2:
                rdmas[j - 2].wait_send()
            if j >= 3:
                ocps[j - 3].wait()
            for b in range(B):
                res_buf[...] = jnp.dot(
                    o_ref[b * S:(b + 1) * S, :], wo_buf[wslot],
                    preferred_element_type=jnp.float32)
                send_buf[sslot, b * S_HALF:(b + 1) * S_HALF, :] = (
                    res_buf[pl.ds(peer_base, S_HALF), :]
                    .astype(jnp.bfloat16))
                acc_buf[aslot, b * S_HALF:(b + 1) * S_HALF, :] = (
                    res_buf[pl.ds(my_base, S_HALF), :])

            rdmas[j] = pltpu.make_async_remote_copy(
                src_ref=send_buf.at[sslot],
                dst_ref=recv_buf.at[j],
                send_sem=send_sems.at[sslot],
                recv_sem=recv_sems.at[j],
                device_id=peer,
                device_id_type=pl.DeviceIdType.MESH,
            )
            rdmas[j].start()

            if j >= 2:
                finalize(j - 2)

        finalize(G - 2)
        finalize(G - 1)
        for j in (G - 3, G - 2, G - 1):
            ocps[j].wait()
        for j in (G - 2, G - 1):
            rdmas[j].wait_send()

    out = pl.pallas_call(
        body,
        out_shape=jax.ShapeDtypeStruct((ROWS, N), jnp.float32),
        in_specs=[
            pl.BlockSpec(memory_space=pltpu.VMEM),
            pl.BlockSpec(memory_space=pl.ANY),
        ],
        out_specs=pl.BlockSpec(memory_space=pltpu.MemorySpace.HBM),
        scratch_shapes=[
            pltpu.VMEM((2, K, NC), jnp.bfloat16),
            pltpu.VMEM((2, ROWS, NC), jnp.bfloat16),
            pltpu.VMEM((G, ROWS, NC), jnp.bfloat16),
            pltpu.VMEM((3, ROWS, NC), jnp.float32),
            pltpu.VMEM((S, NC), jnp.float32),
            pltpu.SemaphoreType.DMA((2,)),
            pltpu.SemaphoreType.DMA((3,)),
            pltpu.SemaphoreType.DMA((2,)),
            pltpu.SemaphoreType.DMA((G,)),
        ],
        compiler_params=pltpu.CompilerParams(
            collective_id=0, vmem_limit_bytes=56 * 1024 * 1024),
    )(O2, Wo2)
    return out.reshape(B, S_HALF, N)


# device time: 260105 ns/iter; 1.0186x vs baseline; 1.0186x over previous
import jax
import jax.numpy as jnp
from jax import lax
from jax.experimental import pallas as pl
from jax.experimental.pallas import tpu as pltpu

B, S, H_LOC, D = 4, 1024, 16, 128
K = H_LOC * D
N = 4096
S_HALF = S // 2
NC = 512
G = N // NC
ROWS = B * S_HALF


def kernel(O, Wo):
    O2 = O.reshape(B * S, K).astype(jnp.bfloat16)
    Wo2 = Wo.astype(jnp.bfloat16)

    def body(o_ref, wo_hbm, out_ref, wo_buf, send_buf, recv_buf, acc_buf,
             wo_sem, out_sem, send_sems, recv_sems):
        my_x = lax.axis_index("x")
        my_y = lax.axis_index("y")
        my_z = lax.axis_index("z")
        peer = (my_x, 1 - my_y, my_z)

        barrier = pltpu.get_barrier_semaphore()
        pl.semaphore_signal(barrier, inc=1, device_id=peer,
                            device_id_type=pl.DeviceIdType.MESH)
        pl.semaphore_wait(barrier, 1)

        my_base = my_y * S_HALF
        peer_base = (1 - my_y) * S_HALF

        def wo_fetch(j):
            return pltpu.make_async_copy(
                wo_hbm.at[:, pl.ds((j % G) * NC, NC)], wo_buf.at[j % 2],
                wo_sem.at[j % 2])

        rdmas = [None] * G
        ocps = [None] * G

        wo_fetch(0).start()
        for j in range(G):
            if j + 1 < G:
                wo_fetch(j + 1).start()
            wo_fetch(j).wait()
            if j >= 4:
                rdmas[j - 4].wait_send()
            for b in range(B):
                lhs = o_ref[pl.ds(b * S + peer_base, S_HALF), :]
                send_buf[j % 4, b * S_HALF:(b + 1) * S_HALF, :] = jnp.dot(
                    lhs, wo_buf[j % 2], preferred_element_type=jnp.float32
                ).astype(jnp.bfloat16)
            rdmas[j] = pltpu.make_async_remote_copy(
                src_ref=send_buf.at[j % 4],
                dst_ref=recv_buf.at[j],
                send_sem=send_sems.at[j % 4],
                recv_sem=recv_sems.at[j],
                device_id=peer,
                device_id_type=pl.DeviceIdType.MESH,
            )
            rdmas[j].start()

        wo_fetch(G).start()
        for j in range(G):
            if j + 1 < G:
                wo_fetch(G + j + 1).start()
            wo_fetch(G + j).wait()
            if j >= 2:
                ocps[j - 2].wait()
            for b in range(B):
                lhs = o_ref[pl.ds(b * S + my_base, S_HALF), :]
                acc_buf[j % 2, b * S_HALF:(b + 1) * S_HALF, :] = jnp.dot(
                    lhs, wo_buf[j % 2], preferred_element_type=jnp.float32)
            rdmas[j].wait_recv()
            acc_buf[j % 2] = acc_buf[j % 2] + recv_buf[j].astype(jnp.float32)
            ocps[j] = pltpu.make_async_copy(
                acc_buf.at[j % 2], out_ref.at[:, pl.ds(j * NC, NC)],
                out_sem.at[j % 2])
            ocps[j].start()

        for j in (G - 2, G - 1):
            ocps[j].wait()
        for j in range(G - 4, G):
            rdmas[j].wait_send()

    out = pl.pallas_call(
        body,
        out_shape=jax.ShapeDtypeStruct((ROWS, N), jnp.float32),
        in_specs=[
            pl.BlockSpec(memory_space=pltpu.VMEM),
            pl.BlockSpec(memory_space=pl.ANY),
        ],
        out_specs=pl.BlockSpec(memory_space=pltpu.MemorySpace.HBM),
        scratch_shapes=[
            pltpu.VMEM((2, K, NC), jnp.bfloat16),
            pltpu.VMEM((4, ROWS, NC), jnp.bfloat16),
            pltpu.VMEM((G, ROWS, NC), jnp.bfloat16),
            pltpu.VMEM((2, ROWS, NC), jnp.float32),
            pltpu.SemaphoreType.DMA((2,)),
            pltpu.SemaphoreType.DMA((2,)),
            pltpu.SemaphoreType.DMA((4,)),
            pltpu.SemaphoreType.DMA((G,)),
        ],
        compiler_params=pltpu.CompilerParams(
            collective_id=0, vmem_limit_bytes=56 * 1024 * 1024),
    )(O2, Wo2)
    return out.reshape(B, S_HALF, N)
